# Initial kernel scaffold; baseline (speedup 1.0000x reference)
import jax
import jax.numpy as jnp
from jax.experimental import pallas as pl


def kernel(loc_data, conf_data, prior_data):
    raise NotImplementedError("write your pallas kernel here")


# speedup vs baseline: 7.9069x; 7.9069x over previous
import functools
import jax
import jax.numpy as jnp
from jax import lax
from jax.experimental import pallas as pl
from jax.experimental.pallas import tpu as pltpu
from jax.experimental.pallas import tpu_sc as plsc

_NUM_CLASSES = 21
_TOP_K = 200
_CONF_THRESH = 0.01
_NMS_THRESH = 0.45
_NUM_PRIORS = 20000
_BATCH = 4

_NUNITS = _BATCH * (_NUM_CLASSES - 1)
_NV = _NUM_PRIORS // 16
_KPAD = 208
_NCV = _KPAD // 16
_PADKEY = -2147483648
_NEGONE_KEY = -1065353217


def _splat(x, n=16):
    return jnp.broadcast_to(jnp.asarray(x, jnp.int32), (n,))


def _key_from_score(s):
    b = lax.bitcast_convert_type(s, jnp.int32)
    return b ^ (lax.shift_right_arithmetic(b, _splat(31)) & _splat(0x7FFFFFFF))


def _score_from_key(k):
    b = k ^ (lax.shift_right_arithmetic(k, _splat(31)) & _splat(0x7FFFFFFF))
    return lax.bitcast_convert_type(b, jnp.float32)


def _sc_detect(conf_ref, tbl_ref, out_ref, scores_v, keys_v, hist_v, gt_v,
               eq_v, selkey_v, selidx_v, skey_v, sidx2_v, rows_v, x1_v, y1_v,
               x2_v, y2_v, ar_v, scs_v, keep_v, outb_v, sem):
    lane = jnp.arange(16, dtype=jnp.int32)
    wid = lax.axis_index("s") * 2 + lax.axis_index("c")

    def process(u):
        b = u // (_NUM_CLASSES - 1)
        pltpu.sync_copy(conf_ref.at[u], scores_v)

        def mk_keys(i, _):
            s = scores_v[pl.ds(i * 16, 16)]
            m = jnp.where(s > _CONF_THRESH, s, jnp.float32(-1.0))
            keys_v[pl.ds(i * 16, 16)] = _key_from_score(m)
            return 0

        lax.fori_loop(0, _NV, mk_keys, 0)

        prefix = jnp.int32(0)
        krem = jnp.int32(_TOP_K)
        for p in range(4):
            shift = 24 - 8 * p

            def zero_hist(j, _):
                hist_v[pl.ds(j * 16, 16)] = jnp.zeros((16,), jnp.int32)
                return 0

            lax.fori_loop(0, 256, zero_hist, 0)

            def histo(i, _, p=p, shift=shift, prefix=prefix):
                k = keys_v[pl.ds(i * 16, 16)]
                if p == 0:
                    dig = lax.shift_right_arithmetic(k, _splat(24)) + 128
                    act = None
                else:
                    act = lax.shift_right_arithmetic(k, _splat(shift + 8)) == _splat(prefix)
                    dig = lax.shift_right_arithmetic(k, _splat(shift)) & _splat(0xFF)
                idx = lane * 256 + dig
                plsc.addupdate_scatter(hist_v, [idx], _splat(1), mask=act)
                return 0

            lax.fori_loop(0, _NV, histo, 0)

            cumhi = jnp.int32(0)
            sel_d = jnp.int32(0)
            above = jnp.int32(0)
            found = jnp.bool_(False)
            for g in range(15, -1, -1):
                tot = jnp.zeros((16,), jnp.int32)
                for l in range(16):
                    tot = tot + hist_v[pl.ds(l * 256 + g * 16, 16)]
                rev = lax.rev(tot, (0,))
                csum = plsc.cumsum(rev)
                cross = (cumhi + csum) >= krem
                firstpos = jnp.min(jnp.where(cross, lane, jnp.int32(16)))
                found_here = jnp.logical_and(firstpos < 16, jnp.logical_not(found))
                onpos = lane == firstpos
                csum_at = jnp.sum(jnp.where(onpos, csum, 0))
                h_at = jnp.sum(jnp.where(onpos, rev, 0))
                d_here = g * 16 + 15 - firstpos
                sel_d = jnp.where(found_here, d_here, sel_d)
                above = jnp.where(found_here, cumhi + csum_at - h_at, above)
                gsum = jnp.sum(tot)
                cumhi = jnp.where(found, cumhi, cumhi + gsum)
                found = jnp.logical_or(found, found_here)
            krem = krem - above
            if p == 0:
                prefix = sel_d - 128
            else:
                prefix = prefix * 256 + sel_d
        tau = prefix
        n_eq_take = krem

        def compact(i, carry):
            ngt, neq = carry
            k = keys_v[pl.ds(i * 16, 16)]
            gidx = i * 16 + lane
            m_gt = k > _splat(tau)
            m_eq = jnp.logical_and(k == _splat(tau), neq < _KPAD)
            plsc.store_compressed(gt_v.at[pl.ds(ngt, 16)], gidx, mask=m_gt)
            plsc.store_compressed(eq_v.at[pl.ds(neq, 16)], gidx, mask=m_eq)
            return (ngt + jnp.sum(m_gt.astype(jnp.int32)),
                    neq + jnp.sum(m_eq.astype(jnp.int32)))

        ngt, _ = lax.fori_loop(0, _NV, compact, (jnp.int32(0), jnp.int32(0)))
        n_eq_take = _TOP_K - ngt

        for c in range(_NCV):
            rr = c * 16 + lane
            selkey_v[pl.ds(c * 16, 16)] = _splat(_PADKEY)
            selidx_v[pl.ds(c * 16, 16)] = _splat(_NUM_PRIORS) + rr
            skey_v[pl.ds(c * 16, 16)] = _splat(_NEGONE_KEY)
            plsc.store_scatter(
                sidx2_v, [rr // 104, rr % 104], jnp.zeros((16,), jnp.int32))

        def fill_gt(c, _):
            off = c * 16 + lane
            valid = off < ngt
            iv = jnp.where(valid, gt_v[pl.ds(c * 16, 16)], 0)
            kv = plsc.load_gather(keys_v, [iv])
            plsc.store_scatter(selkey_v, [off], kv, mask=valid)
            plsc.store_scatter(selidx_v, [off], iv, mask=valid)
            return 0

        lax.fori_loop(0, _NCV, fill_gt, 0)

        def fill_eq(c, _):
            off = c * 16 + lane
            valid = off < n_eq_take
            iv = jnp.where(valid, eq_v[pl.ds(c * 16, 16)], 0)
            kv = plsc.load_gather(keys_v, [iv])
            pos = ngt + off
            plsc.store_scatter(selkey_v, [pos], kv, mask=valid)
            plsc.store_scatter(selidx_v, [pos], iv, mask=valid)
            return 0

        lax.fori_loop(0, _NCV, fill_eq, 0)

        def rank_body(j, acc):
            kj = plsc.load_gather(selkey_v, [_splat(j)])
            ij = plsc.load_gather(selidx_v, [_splat(j)])
            out = []
            for c in range(_NCV):
                kc = selkey_v[pl.ds(c * 16, 16)]
                ic = selidx_v[pl.ds(c * 16, 16)]
                beats = jnp.logical_or(
                    kj > kc, jnp.logical_and(kj == kc, ij < ic))
                out.append(acc[c] + beats.astype(jnp.int32))
            return tuple(out)

        ranks = lax.fori_loop(
            0, _TOP_K, rank_body,
            tuple(jnp.zeros((16,), jnp.int32) for _ in range(_NCV)))
        for c in range(_NCV):
            r = ranks[c]
            m = r < _TOP_K
            kc = selkey_v[pl.ds(c * 16, 16)]
            ic = selidx_v[pl.ds(c * 16, 16)]
            plsc.store_scatter(skey_v, [r], kc, mask=m)
            plsc.store_scatter(
                sidx2_v, [r // 104, r % 104],
                ic + b * _NUM_PRIORS, mask=m)

        for c in range(_NCV):
            scs_v[pl.ds(c * 16, 16)] = _score_from_key(skey_v[pl.ds(c * 16, 16)])

        for h in range(2):
            pltpu.async_copy(
                tbl_ref.at[sidx2_v.at[h]],
                rows_v.at[pl.ds(h * 104, 104)], sem).wait()
        for c in range(_NCV):
            rr = c * 16 + lane

            def comp(j):
                return plsc.load_gather(rows_v, [rr, _splat(j)])

            l0, l1, l2, l3 = comp(0), comp(1), comp(2), comp(3)
            p0, p1, p2, p3 = comp(4), comp(5), comp(6), comp(7)
            cx = p0 + l0 * jnp.float32(_VAR0) * p2
            cy = p1 + l1 * jnp.float32(_VAR0) * p3
            w = p2 * jnp.exp(l2 * jnp.float32(_VAR1))
            h_ = p3 * jnp.exp(l3 * jnp.float32(_VAR1))
            x1 = cx - w / 2.0
            y1 = cy - h_ / 2.0
            x2 = x1 + w
            y2 = y1 + h_
            x1_v[pl.ds(c * 16, 16)] = x1
            y1_v[pl.ds(c * 16, 16)] = y1
            x2_v[pl.ds(c * 16, 16)] = x2
            y2_v[pl.ds(c * 16, 16)] = y2
            ar_v[pl.ds(c * 16, 16)] = (x2 - x1) * (y2 - y1)
            keep_v[pl.ds(c * 16, 16)] = (
                scs_v[pl.ds(c * 16, 16)] > _CONF_THRESH).astype(jnp.int32)

        def nms_body(i, _):
            spl = _splat(i)
            ki = plsc.load_gather(keep_v, [spl]) > 0
            x1i = plsc.load_gather(x1_v, [spl])
            y1i = plsc.load_gather(y1_v, [spl])
            x2i = plsc.load_gather(x2_v, [spl])
            y2i = plsc.load_gather(y2_v, [spl])
            ari = plsc.load_gather(ar_v, [spl])
            for c in range(_NCV):
                sl = pl.ds(c * 16, 16)
                xx1 = jnp.maximum(x1i, x1_v[sl])
                yy1 = jnp.maximum(y1i, y1_v[sl])
                xx2 = jnp.minimum(x2i, x2_v[sl])
                yy2 = jnp.minimum(y2i, y2_v[sl])
                w = jnp.maximum(xx2 - xx1, jnp.float32(0.0))
                h = jnp.maximum(yy2 - yy1, jnp.float32(0.0))
                inter = w * h
                union = ari + ar_v[sl] - inter
                iou = inter / jnp.maximum(union, jnp.float32(1e-12))
                sup = jnp.logical_and(iou > _NMS_THRESH, (c * 16 + lane) > i)
                sup = jnp.logical_and(sup, ki)
                keep_v[sl] = jnp.where(sup, 0, keep_v[sl])
            return 0

        lax.fori_loop(0, _TOP_K, nms_body, 0)

        def zero_out(j, _):
            outb_v[pl.ds(j * 16, 16)] = jnp.zeros((16,), jnp.float32)
            return 0

        lax.fori_loop(0, (_TOP_K * 5 + 40) // 16, zero_out, 0)
        cnt = jnp.int32(0)
        for c in range(_NCV):
            sl = pl.ds(c * 16, 16)
            kc = keep_v[sl]
            csum = plsc.cumsum(kc)
            pos = cnt + csum - kc
            m = kc > 0
            base = pos * 5
            plsc.store_scatter(outb_v, [base], scs_v[sl], mask=m)
            plsc.store_scatter(outb_v, [base + 1], x1_v[sl], mask=m)
            plsc.store_scatter(outb_v, [base + 2], y1_v[sl], mask=m)
            plsc.store_scatter(outb_v, [base + 3], x2_v[sl], mask=m)
            plsc.store_scatter(outb_v, [base + 4], y2_v[sl], mask=m)
            cnt = cnt + jnp.sum(kc)
        pltpu.sync_copy(outb_v, out_ref.at[u])

    def round_body(r, _):
        u = wid + r * 32

        @pl.when(u < _NUNITS)
        def _():
            process(u)

        return 0

    lax.fori_loop(0, 3, round_body, 0)


_VAR0 = 0.1
_VAR1 = 0.2
_OUTROW = _TOP_K * 5 + 40


@jax.jit
def kernel(loc_data, conf_data, prior_data):
    conf_t = jnp.transpose(conf_data, (0, 2, 1))[:, 1:, :].reshape(
        _NUNITS, _NUM_PRIORS)
    tbl = jnp.concatenate(
        [loc_data.reshape(_BATCH * _NUM_PRIORS, 4),
         jnp.tile(prior_data, (_BATCH, 1))], axis=1)

    mesh = plsc.VectorSubcoreMesh(core_axis_name="c", subcore_axis_name="s")
    out = pl.kernel(
        _sc_detect,
        out_type=jax.ShapeDtypeStruct((_NUNITS, _OUTROW), jnp.float32),
        mesh=mesh,
        compiler_params=pltpu.CompilerParams(
            needs_layout_passes=False, use_tc_tiling_on_sc=False),
        scratch_types=[
            pltpu.VMEM((_NUM_PRIORS,), jnp.float32),
            pltpu.VMEM((_NUM_PRIORS,), jnp.int32),
            pltpu.VMEM((4096,), jnp.int32),
            pltpu.VMEM((224,), jnp.int32),
            pltpu.VMEM((224,), jnp.int32),
            pltpu.VMEM((_KPAD,), jnp.int32),
            pltpu.VMEM((_KPAD,), jnp.int32),
            pltpu.VMEM((_KPAD,), jnp.int32),
            pltpu.VMEM((2, 104), jnp.int32),
            pltpu.VMEM((_KPAD, 8), jnp.float32),
            pltpu.VMEM((_KPAD,), jnp.float32),
            pltpu.VMEM((_KPAD,), jnp.float32),
            pltpu.VMEM((_KPAD,), jnp.float32),
            pltpu.VMEM((_KPAD,), jnp.float32),
            pltpu.VMEM((_KPAD,), jnp.float32),
            pltpu.VMEM((_KPAD,), jnp.float32),
            pltpu.VMEM((_KPAD,), jnp.int32),
            pltpu.VMEM((_OUTROW,), jnp.float32),
            pltpu.SemaphoreType.DMA,
        ],
    )(conf_t, tbl)

    dets = out[:, : _TOP_K * 5].reshape(_BATCH, _NUM_CLASSES - 1, _TOP_K, 5)
    zeros0 = jnp.zeros((_BATCH, 1, _TOP_K, 5), jnp.float32)
    return jnp.concatenate([zeros0, dets], axis=1)


# speedup vs baseline: 10.3958x; 1.3148x over previous
import jax
import jax.numpy as jnp
from jax import lax
from jax.experimental import pallas as pl
from jax.experimental.pallas import tpu as pltpu
from jax.experimental.pallas import tpu_sc as plsc

_NUM_CLASSES = 21
_TOP_K = 200
_CONF_THRESH = 0.01
_NMS_THRESH = 0.45
_VAR0 = 0.1
_VAR1 = 0.2
_NUM_PRIORS = 20000
_BATCH = 4

_NUNITS = _BATCH * (_NUM_CLASSES - 1)
_NV = _NUM_PRIORS // 16
_KPAD = 208
_NCV = _KPAD // 16
_PADKEY = -2147483648
_NEGONE_KEY = -1065353217
_OUTROW = _TOP_K * 5 + 40


def _splat(x, n=16):
    return jnp.broadcast_to(jnp.asarray(x, jnp.int32), (n,))


def _sra(x, n):
    return lax.shift_right_arithmetic(x, _splat(n))


def _key_from_score(s):
    b = lax.bitcast_convert_type(s, jnp.int32)
    return b ^ (_sra(b, 31) & _splat(0x7FFFFFFF))


def _score_from_key(k):
    b = k ^ (_sra(k, 31) & _splat(0x7FFFFFFF))
    return lax.bitcast_convert_type(b, jnp.float32)


def _popcnt(mask):
    return plsc.all_reduce_population_count(mask)


def _sc_detect(conf_ref, tbl_ref, out_ref, scores_v, keys_v, rkey_v, ridx_v,
               hist_v, gt_v, eq_v, selkey_v, selidx_v, skey_v, sidx2_v,
               rows_v, x1_v, y1_v, x2_v, y2_v, ar_v, scs_v, keep_v, outb_v,
               sem):
    lane = jnp.arange(16, dtype=jnp.int32)
    wid = lax.axis_index("s") * 2 + lax.axis_index("c")

    def zero_hist():
        def zh(j, _):
            for t in range(4):
                hist_v[pl.ds((4 * j + t) * 16, 16)] = jnp.zeros((16,), jnp.int32)
            return 0

        lax.fori_loop(0, 64, zh, 0)

    def search(krem):
        cumhi = jnp.int32(0)
        sel_d = jnp.int32(0)
        above = jnp.int32(0)
        found = jnp.bool_(False)
        for g in range(15, -1, -1):
            tot = jnp.zeros((16,), jnp.int32)
            for l in range(16):
                tot = tot + hist_v[pl.ds(l * 256 + g * 16, 16)]
            rev = lax.rev(tot, (0,))
            csum = plsc.cumsum(rev)
            cross = (cumhi + csum) >= krem
            firstpos = jnp.min(jnp.where(cross, lane, jnp.int32(16)))
            found_here = jnp.logical_and(firstpos < 16, jnp.logical_not(found))
            onpos = lane == firstpos
            csum_at = jnp.sum(jnp.where(onpos, csum, 0))
            h_at = jnp.sum(jnp.where(onpos, rev, 0))
            d_here = g * 16 + 15 - firstpos
            sel_d = jnp.where(found_here, d_here, sel_d)
            above = jnp.where(found_here, cumhi + csum_at - h_at, above)
            gsum = jnp.sum(tot)
            cumhi = jnp.where(found, cumhi, cumhi + gsum)
            found = jnp.logical_or(found, found_here)
        return sel_d, above

    def process(u):
        b = u // (_NUM_CLASSES - 1)
        pltpu.sync_copy(conf_ref.at[u], scores_v)

        zero_hist()

        def scan1(i, _):
            for t in range(2):
                sl = pl.ds((2 * i + t) * 16, 16)
                s = scores_v[sl]
                m = jnp.where(s > _CONF_THRESH, s, jnp.float32(-1.0))
                k = _key_from_score(m)
                keys_v[sl] = k
                dig = _sra(k, 24) + 128
                plsc.addupdate_scatter(hist_v, [lane * 256 + dig], _splat(1))
            return 0

        lax.fori_loop(0, _NV // 2, scan1, 0)
        krem = jnp.int32(_TOP_K)
        d0, above = search(krem)
        krem = krem - above
        prefix8 = d0 - 128

        zero_hist()

        def scan2(i, _):
            for t in range(2):
                k = keys_v[pl.ds((2 * i + t) * 16, 16)]
                act = _sra(k, 24) == prefix8
                dig = _sra(k, 16) & _splat(0xFF)
                plsc.addupdate_scatter(
                    hist_v, [lane * 256 + dig], _splat(1), mask=act)
            return 0

        lax.fori_loop(0, _NV // 2, scan2, 0)
        d1, above = search(krem)
        krem = krem - above
        prefix16 = prefix8 * 256 + d1

        def scan3(i, carry):
            goff, roff = carry
            for t in range(2):
                base = (2 * i + t) * 16
                k = keys_v[pl.ds(base, 16)]
                gidx = base + lane
                hi = _sra(k, 16)
                m_gt = hi > prefix16
                m_eq = hi == prefix16
                pos_g = goff + plsc.cumsum(m_gt.astype(jnp.int32)) - m_gt
                plsc.store_scatter(gt_v, [pos_g], gidx, mask=m_gt)
                pos_r = roff + plsc.cumsum(m_eq.astype(jnp.int32)) - m_eq
                plsc.store_scatter(rkey_v, [pos_r], k, mask=m_eq)
                plsc.store_scatter(ridx_v, [pos_r], gidx, mask=m_eq)
                goff = goff + _popcnt(m_gt)
                roff = roff + _popcnt(m_eq)
            return goff, roff

        goff, roff = lax.fori_loop(
            0, _NV // 2, scan3,
            (jnp.zeros((16,), jnp.int32), jnp.zeros((16,), jnp.int32)))
        nref = jnp.max(roff)
        nact = (nref + 15) // 16

        zero_hist()

        def p3(i, _):
            k = rkey_v[pl.ds(i * 16, 16)]
            act = (i * 16 + lane) < roff
            dig = _sra(k, 8) & _splat(0xFF)
            plsc.addupdate_scatter(hist_v, [lane * 256 + dig], _splat(1),
                                   mask=act)
            return 0

        lax.fori_loop(0, nact, p3, 0)
        d2, above = search(krem)
        krem = krem - above
        prefix24 = prefix16 * 256 + d2

        zero_hist()

        def p4(i, _):
            k = rkey_v[pl.ds(i * 16, 16)]
            act = jnp.logical_and((i * 16 + lane) < roff,
                                  _sra(k, 8) == prefix24)
            dig = k & _splat(0xFF)
            plsc.addupdate_scatter(hist_v, [lane * 256 + dig], _splat(1),
                                   mask=act)
            return 0

        lax.fori_loop(0, nact, p4, 0)
        d3, above = search(krem)
        krem = krem - above
        tau = prefix24 * 256 + d3
        n_eq_take = krem

        def fcomp(i, carry):
            goff, eoff = carry
            sl = pl.ds(i * 16, 16)
            k = rkey_v[sl]
            iv = ridx_v[sl]
            tail = (i * 16 + lane) < roff
            m_gt = jnp.logical_and(k > _splat(tau), tail)
            m_eq = jnp.logical_and(
                jnp.logical_and(k == _splat(tau), tail), eoff < _KPAD)
            pos_g = goff + plsc.cumsum(m_gt.astype(jnp.int32)) - m_gt
            plsc.store_scatter(gt_v, [pos_g], iv, mask=m_gt)
            pos_e = eoff + plsc.cumsum(m_eq.astype(jnp.int32)) - m_eq
            plsc.store_scatter(eq_v, [pos_e], iv, mask=m_eq)
            return goff + _popcnt(m_gt), eoff + _popcnt(m_eq)

        ngt_v, _ = lax.fori_loop(
            0, nact, fcomp, (goff, jnp.zeros((16,), jnp.int32)))

        for c in range(_NCV):
            rr = c * 16 + lane
            selkey_v[pl.ds(c * 16, 16)] = _splat(_PADKEY)
            selidx_v[pl.ds(c * 16, 16)] = _splat(_NUM_PRIORS) + rr
            skey_v[pl.ds(c * 16, 16)] = _splat(_NEGONE_KEY)
            plsc.store_scatter(
                sidx2_v, [rr // 104, rr % 104], jnp.zeros((16,), jnp.int32))

        def fill_gt(c, _):
            off = c * 16 + lane
            valid = off < ngt_v
            iv = jnp.where(valid, gt_v[pl.ds(c * 16, 16)], 0)
            kv = plsc.load_gather(keys_v, [iv])
            plsc.store_scatter(selkey_v, [off], kv, mask=valid)
            plsc.store_scatter(selidx_v, [off], iv, mask=valid)
            return 0

        lax.fori_loop(0, _NCV, fill_gt, 0)

        def fill_eq(c, _):
            off = c * 16 + lane
            valid = off < n_eq_take
            iv = jnp.where(valid, eq_v[pl.ds(c * 16, 16)], 0)
            kv = plsc.load_gather(keys_v, [iv])
            pos = ngt_v + off
            plsc.store_scatter(selkey_v, [pos], kv, mask=valid)
            plsc.store_scatter(selidx_v, [pos], iv, mask=valid)
            return 0

        lax.fori_loop(0, _NCV, fill_eq, 0)

        def rank_body(j, acc):
            kj = plsc.load_gather(selkey_v, [_splat(j)])
            ij = plsc.load_gather(selidx_v, [_splat(j)])
            out = []
            for c in range(_NCV):
                kc = selkey_v[pl.ds(c * 16, 16)]
                ic = selidx_v[pl.ds(c * 16, 16)]
                beats = jnp.logical_or(
                    kj > kc, jnp.logical_and(kj == kc, ij < ic))
                out.append(acc[c] + beats.astype(jnp.int32))
            return tuple(out)

        ranks = lax.fori_loop(
            0, _TOP_K, rank_body,
            tuple(jnp.zeros((16,), jnp.int32) for _ in range(_NCV)))
        for c in range(_NCV):
            r = ranks[c]
            m = r < _TOP_K
            kc = selkey_v[pl.ds(c * 16, 16)]
            ic = selidx_v[pl.ds(c * 16, 16)]
            plsc.store_scatter(skey_v, [r], kc, mask=m)
            plsc.store_scatter(
                sidx2_v, [r // 104, r % 104], ic + b * _NUM_PRIORS, mask=m)

        for c in range(_NCV):
            scs_v[pl.ds(c * 16, 16)] = _score_from_key(skey_v[pl.ds(c * 16, 16)])

        for h in range(2):
            pltpu.async_copy(
                tbl_ref.at[sidx2_v.at[h]],
                rows_v.at[pl.ds(h * 104, 104)], sem).wait()
        for c in range(_NCV):
            rr = c * 16 + lane

            def comp(j):
                return plsc.load_gather(rows_v, [rr, _splat(j)])

            l0, l1, l2, l3 = comp(0), comp(1), comp(2), comp(3)
            p0, p1, p2, p3 = comp(4), comp(5), comp(6), comp(7)
            cx = p0 + l0 * jnp.float32(_VAR0) * p2
            cy = p1 + l1 * jnp.float32(_VAR0) * p3
            w = p2 * jnp.exp(l2 * jnp.float32(_VAR1))
            h_ = p3 * jnp.exp(l3 * jnp.float32(_VAR1))
            x1 = cx - w / 2.0
            y1 = cy - h_ / 2.0
            x2 = x1 + w
            y2 = y1 + h_
            x1_v[pl.ds(c * 16, 16)] = x1
            y1_v[pl.ds(c * 16, 16)] = y1
            x2_v[pl.ds(c * 16, 16)] = x2
            y2_v[pl.ds(c * 16, 16)] = y2
            ar_v[pl.ds(c * 16, 16)] = (x2 - x1) * (y2 - y1)
            keep_v[pl.ds(c * 16, 16)] = (
                scs_v[pl.ds(c * 16, 16)] > _CONF_THRESH).astype(jnp.int32)

        def nms_body(i, _):
            spl = _splat(i)
            ki = plsc.load_gather(keep_v, [spl]) > 0
            x1i = plsc.load_gather(x1_v, [spl])
            y1i = plsc.load_gather(y1_v, [spl])
            x2i = plsc.load_gather(x2_v, [spl])
            y2i = plsc.load_gather(y2_v, [spl])
            ari = plsc.load_gather(ar_v, [spl])
            for c in range(_NCV):
                sl = pl.ds(c * 16, 16)
                xx1 = jnp.maximum(x1i, x1_v[sl])
                yy1 = jnp.maximum(y1i, y1_v[sl])
                xx2 = jnp.minimum(x2i, x2_v[sl])
                yy2 = jnp.minimum(y2i, y2_v[sl])
                w = jnp.maximum(xx2 - xx1, jnp.float32(0.0))
                h = jnp.maximum(yy2 - yy1, jnp.float32(0.0))
                inter = w * h
                union = ari + ar_v[sl] - inter
                iou = inter / jnp.maximum(union, jnp.float32(1e-12))
                sup = jnp.logical_and(iou > _NMS_THRESH, (c * 16 + lane) > i)
                sup = jnp.logical_and(sup, ki)
                keep_v[sl] = jnp.where(sup, 0, keep_v[sl])
            return 0

        lax.fori_loop(0, _TOP_K, nms_body, 0)

        def zero_out(j, _):
            outb_v[pl.ds(j * 16, 16)] = jnp.zeros((16,), jnp.float32)
            return 0

        lax.fori_loop(0, _OUTROW // 16, zero_out, 0)
        cnt = jnp.int32(0)
        for c in range(_NCV):
            sl = pl.ds(c * 16, 16)
            kc = keep_v[sl]
            csum = plsc.cumsum(kc)
            pos = cnt + csum - kc
            m = kc > 0
            base = pos * 5
            plsc.store_scatter(outb_v, [base], scs_v[sl], mask=m)
            plsc.store_scatter(outb_v, [base + 1], x1_v[sl], mask=m)
            plsc.store_scatter(outb_v, [base + 2], y1_v[sl], mask=m)
            plsc.store_scatter(outb_v, [base + 3], x2_v[sl], mask=m)
            plsc.store_scatter(outb_v, [base + 4], y2_v[sl], mask=m)
            cnt = cnt + jnp.sum(kc)
        pltpu.sync_copy(outb_v, out_ref.at[u])

    def round_body(r, _):
        u = wid + r * 32

        @pl.when(u < _NUNITS)
        def _():
            process(u)

        return 0

    lax.fori_loop(0, 3, round_body, 0)


@jax.jit
def kernel(loc_data, conf_data, prior_data):
    conf_t = jnp.transpose(conf_data, (0, 2, 1))[:, 1:, :].reshape(
        _NUNITS, _NUM_PRIORS)
    tbl = jnp.concatenate(
        [loc_data.reshape(_BATCH * _NUM_PRIORS, 4),
         jnp.tile(prior_data, (_BATCH, 1))], axis=1)

    mesh = plsc.VectorSubcoreMesh(core_axis_name="c", subcore_axis_name="s")
    out = pl.kernel(
        _sc_detect,
        out_type=jax.ShapeDtypeStruct((_NUNITS, _OUTROW), jnp.float32),
        mesh=mesh,
        compiler_params=pltpu.CompilerParams(
            needs_layout_passes=False, use_tc_tiling_on_sc=False),
        scratch_types=[
            pltpu.VMEM((_NUM_PRIORS,), jnp.float32),
            pltpu.VMEM((_NUM_PRIORS,), jnp.int32),
            pltpu.VMEM((_NUM_PRIORS,), jnp.int32),
            pltpu.VMEM((_NUM_PRIORS,), jnp.int32),
            pltpu.VMEM((4096,), jnp.int32),
            pltpu.VMEM((224,), jnp.int32),
            pltpu.VMEM((224,), jnp.int32),
            pltpu.VMEM((_KPAD,), jnp.int32),
            pltpu.VMEM((_KPAD,), jnp.int32),
            pltpu.VMEM((_KPAD,), jnp.int32),
            pltpu.VMEM((2, 104), jnp.int32),
            pltpu.VMEM((_KPAD, 8), jnp.float32),
            pltpu.VMEM((_KPAD,), jnp.float32),
            pltpu.VMEM((_KPAD,), jnp.float32),
            pltpu.VMEM((_KPAD,), jnp.float32),
            pltpu.VMEM((_KPAD,), jnp.float32),
            pltpu.VMEM((_KPAD,), jnp.float32),
            pltpu.VMEM((_KPAD,), jnp.float32),
            pltpu.VMEM((_KPAD,), jnp.int32),
            pltpu.VMEM((_OUTROW,), jnp.float32),
            pltpu.SemaphoreType.DMA,
        ],
    )(conf_t, tbl)

    dets = out[:, : _TOP_K * 5].reshape(_BATCH, _NUM_CLASSES - 1, _TOP_K, 5)
    zeros0 = jnp.zeros((_BATCH, 1, _TOP_K, 5), jnp.float32)
    return jnp.concatenate([zeros0, dets], axis=1)


# speedup vs baseline: 10.6275x; 1.0223x over previous
import jax
import jax.numpy as jnp
from jax import lax
from jax.experimental import pallas as pl
from jax.experimental.pallas import tpu as pltpu
from jax.experimental.pallas import tpu_sc as plsc

_NUM_CLASSES = 21
_TOP_K = 200
_CONF_THRESH = 0.01
_NMS_THRESH = 0.45
_VAR0 = 0.1
_VAR1 = 0.2
_NUM_PRIORS = 20000
_BATCH = 4

_NUNITS = _BATCH * (_NUM_CLASSES - 1)
_NPAD = 20480
_NV = _NPAD // 16
_KPAD = 208
_NCV = _KPAD // 16
_PADKEY = -2147483648
_NEGONE_KEY = -1065353217
_OUTROW = _TOP_K * 5 + 40


def _splat(x, n=16):
    return jnp.broadcast_to(jnp.asarray(x, jnp.int32), (n,))


def _sra(x, n):
    return lax.shift_right_arithmetic(x, _splat(n))


def _key_from_score(s):
    b = lax.bitcast_convert_type(s, jnp.int32)
    return b ^ (_sra(b, 31) & _splat(0x7FFFFFFF))


def _score_from_key(k):
    b = k ^ (_sra(k, 31) & _splat(0x7FFFFFFF))
    return lax.bitcast_convert_type(b, jnp.float32)


def _popcnt(mask):
    return plsc.all_reduce_population_count(mask)


def _sc_detect(conf_ref, tbl_ref, out_ref, scores_v, keys_v, rkey_v, ridx_v,
               hist_v, gt_v, eq_v, selkey_v, selidx_v, skey_v, sidx2_v,
               rows_v, x1_v, y1_v, x2_v, y2_v, ar_v, scs_v, keep_v, outb_v,
               sem):
    lane = jnp.arange(16, dtype=jnp.int32)
    wid = lax.axis_index("s") * 2 + lax.axis_index("c")

    def zero_hist():
        def zh(j, _):
            for t in range(4):
                hist_v[pl.ds((4 * j + t) * 16, 16)] = jnp.zeros((16,), jnp.int32)
            return 0

        lax.fori_loop(0, 64, zh, 0)

    def search(krem):
        cumhi = jnp.int32(0)
        sel_d = jnp.int32(0)
        above = jnp.int32(0)
        found = jnp.bool_(False)
        for g in range(15, -1, -1):
            tot = jnp.zeros((16,), jnp.int32)
            for l in range(16):
                tot = tot + hist_v[pl.ds(l * 256 + g * 16, 16)]
            rev = lax.rev(tot, (0,))
            csum = plsc.cumsum(rev)
            cross = (cumhi + csum) >= krem
            firstpos = jnp.min(jnp.where(cross, lane, jnp.int32(16)))
            found_here = jnp.logical_and(firstpos < 16, jnp.logical_not(found))
            onpos = lane == firstpos
            csum_at = jnp.sum(jnp.where(onpos, csum, 0))
            h_at = jnp.sum(jnp.where(onpos, rev, 0))
            d_here = g * 16 + 15 - firstpos
            sel_d = jnp.where(found_here, d_here, sel_d)
            above = jnp.where(found_here, cumhi + csum_at - h_at, above)
            gsum = jnp.sum(tot)
            cumhi = jnp.where(found, cumhi, cumhi + gsum)
            found = jnp.logical_or(found, found_here)
        return sel_d, above

    def process(u):
        b = u // (_NUM_CLASSES - 1)
        pltpu.sync_copy(conf_ref.at[u], scores_v.at[pl.ds(0, _NUM_PRIORS)])
        for t in range((_NPAD - _NUM_PRIORS) // 16):
            scores_v[pl.ds(_NUM_PRIORS + t * 16, 16)] = jnp.full(
                (16,), -1.0, jnp.float32)

        zero_hist()

        def scan1(i, _):
            for t in range(4):
                sl = pl.ds((4 * i + t) * 16, 16)
                s = scores_v[sl]
                m = jnp.where(s > _CONF_THRESH, s, jnp.float32(-1.0))
                k = _key_from_score(m)
                keys_v[sl] = k
                dig = _sra(k, 24) + 128
                plsc.addupdate_scatter(hist_v, [lane * 256 + dig], _splat(1))
            return 0

        lax.fori_loop(0, _NV // 4, scan1, 0)
        krem = jnp.int32(_TOP_K)
        d0, above = search(krem)
        krem = krem - above
        prefix8 = d0 - 128

        zero_hist()

        def scan2(i, _):
            for t in range(4):
                k = keys_v[pl.ds((4 * i + t) * 16, 16)]
                act = _sra(k, 24) == prefix8
                dig = _sra(k, 16) & _splat(0xFF)
                plsc.addupdate_scatter(
                    hist_v, [lane * 256 + dig], _splat(1), mask=act)
            return 0

        lax.fori_loop(0, _NV // 4, scan2, 0)
        d1, above = search(krem)
        krem = krem - above
        prefix16 = prefix8 * 256 + d1

        def scan3(i, carry):
            goff, roff = carry
            for t in range(4):
                base = (4 * i + t) * 16
                k = keys_v[pl.ds(base, 16)]
                gidx = base + lane
                hi = _sra(k, 16)
                m_gt = hi > prefix16
                m_eq = hi == prefix16
                pos_g = goff + plsc.cumsum(m_gt.astype(jnp.int32)) - m_gt
                plsc.store_scatter(gt_v, [pos_g], gidx, mask=m_gt)
                pos_r = roff + plsc.cumsum(m_eq.astype(jnp.int32)) - m_eq
                plsc.store_scatter(rkey_v, [pos_r], k, mask=m_eq)
                plsc.store_scatter(ridx_v, [pos_r], gidx, mask=m_eq)
                goff = goff + _popcnt(m_gt)
                roff = roff + _popcnt(m_eq)
            return goff, roff

        goff, roff = lax.fori_loop(
            0, _NV // 4, scan3,
            (jnp.zeros((16,), jnp.int32), jnp.zeros((16,), jnp.int32)))
        nref = jnp.max(roff)
        nact = (nref + 15) // 16

        zero_hist()

        def p3(i, _):
            k = rkey_v[pl.ds(i * 16, 16)]
            act = (i * 16 + lane) < roff
            dig = _sra(k, 8) & _splat(0xFF)
            plsc.addupdate_scatter(hist_v, [lane * 256 + dig], _splat(1),
                                   mask=act)
            return 0

        lax.fori_loop(0, nact, p3, 0)
        d2, above = search(krem)
        krem = krem - above
        prefix24 = prefix16 * 256 + d2

        zero_hist()

        def p4(i, _):
            k = rkey_v[pl.ds(i * 16, 16)]
            act = jnp.logical_and((i * 16 + lane) < roff,
                                  _sra(k, 8) == prefix24)
            dig = k & _splat(0xFF)
            plsc.addupdate_scatter(hist_v, [lane * 256 + dig], _splat(1),
                                   mask=act)
            return 0

        lax.fori_loop(0, nact, p4, 0)
        d3, above = search(krem)
        krem = krem - above
        tau = prefix24 * 256 + d3
        n_eq_take = krem

        def fcomp(i, carry):
            goff, eoff = carry
            sl = pl.ds(i * 16, 16)
            k = rkey_v[sl]
            iv = ridx_v[sl]
            tail = (i * 16 + lane) < roff
            m_gt = jnp.logical_and(k > _splat(tau), tail)
            m_eq = jnp.logical_and(
                jnp.logical_and(k == _splat(tau), tail), eoff < _KPAD)
            pos_g = goff + plsc.cumsum(m_gt.astype(jnp.int32)) - m_gt
            plsc.store_scatter(gt_v, [pos_g], iv, mask=m_gt)
            pos_e = eoff + plsc.cumsum(m_eq.astype(jnp.int32)) - m_eq
            plsc.store_scatter(eq_v, [pos_e], iv, mask=m_eq)
            return goff + _popcnt(m_gt), eoff + _popcnt(m_eq)

        ngt_v, _ = lax.fori_loop(
            0, nact, fcomp, (goff, jnp.zeros((16,), jnp.int32)))

        for c in range(_NCV):
            rr = c * 16 + lane
            selkey_v[pl.ds(c * 16, 16)] = _splat(_PADKEY)
            selidx_v[pl.ds(c * 16, 16)] = _splat(_NUM_PRIORS) + rr
            skey_v[pl.ds(c * 16, 16)] = _splat(_NEGONE_KEY)
            plsc.store_scatter(
                sidx2_v, [rr // 104, rr % 104], jnp.zeros((16,), jnp.int32))

        def fill_gt(c, _):
            off = c * 16 + lane
            valid = off < ngt_v
            iv = jnp.where(valid, gt_v[pl.ds(c * 16, 16)], 0)
            kv = plsc.load_gather(keys_v, [iv])
            plsc.store_scatter(selkey_v, [off], kv, mask=valid)
            plsc.store_scatter(selidx_v, [off], iv, mask=valid)
            return 0

        lax.fori_loop(0, _NCV, fill_gt, 0)

        def fill_eq(c, _):
            off = c * 16 + lane
            valid = off < n_eq_take
            iv = jnp.where(valid, eq_v[pl.ds(c * 16, 16)], 0)
            kv = plsc.load_gather(keys_v, [iv])
            pos = ngt_v + off
            plsc.store_scatter(selkey_v, [pos], kv, mask=valid)
            plsc.store_scatter(selidx_v, [pos], iv, mask=valid)
            return 0

        lax.fori_loop(0, _NCV, fill_eq, 0)

        kcs = tuple(selkey_v[pl.ds(c * 16, 16)] for c in range(_NCV))
        ics = tuple(selidx_v[pl.ds(c * 16, 16)] for c in range(_NCV))

        def rank_body(j, acc):
            kj = plsc.load_gather(selkey_v, [_splat(j)])
            ij = plsc.load_gather(selidx_v, [_splat(j)])
            out = []
            for c in range(_NCV):
                beats = jnp.logical_or(
                    kj > kcs[c], jnp.logical_and(kj == kcs[c], ij < ics[c]))
                out.append(acc[c] + beats.astype(jnp.int32))
            return tuple(out)

        ranks = lax.fori_loop(
            0, _TOP_K, rank_body,
            tuple(jnp.zeros((16,), jnp.int32) for _ in range(_NCV)))
        for c in range(_NCV):
            r = ranks[c]
            m = r < _TOP_K
            plsc.store_scatter(skey_v, [r], kcs[c], mask=m)
            plsc.store_scatter(
                sidx2_v, [r // 104, r % 104], ics[c] + b * _NUM_PRIORS, mask=m)

        for c in range(_NCV):
            scs_v[pl.ds(c * 16, 16)] = _score_from_key(skey_v[pl.ds(c * 16, 16)])

        for h in range(2):
            pltpu.async_copy(
                tbl_ref.at[sidx2_v.at[h]],
                rows_v.at[pl.ds(h * 104, 104)], sem).wait()
        for c in range(_NCV):
            rr = c * 16 + lane

            def comp(j):
                return plsc.load_gather(rows_v, [rr, _splat(j)])

            l0, l1, l2, l3 = comp(0), comp(1), comp(2), comp(3)
            p0, p1, p2, p3 = comp(4), comp(5), comp(6), comp(7)
            cx = p0 + l0 * jnp.float32(_VAR0) * p2
            cy = p1 + l1 * jnp.float32(_VAR0) * p3
            w = p2 * jnp.exp(l2 * jnp.float32(_VAR1))
            h_ = p3 * jnp.exp(l3 * jnp.float32(_VAR1))
            x1 = cx - w / 2.0
            y1 = cy - h_ / 2.0
            x2 = x1 + w
            y2 = y1 + h_
            x1_v[pl.ds(c * 16, 16)] = x1
            y1_v[pl.ds(c * 16, 16)] = y1
            x2_v[pl.ds(c * 16, 16)] = x2
            y2_v[pl.ds(c * 16, 16)] = y2
            ar_v[pl.ds(c * 16, 16)] = (x2 - x1) * (y2 - y1)
            keep_v[pl.ds(c * 16, 16)] = (
                scs_v[pl.ds(c * 16, 16)] > _CONF_THRESH).astype(jnp.int32)

        for cb in range(_NCV):
            def nms_body(ii, _, cb=cb):
                i = cb * 16 + ii
                spl = _splat(i)
                ki = plsc.load_gather(keep_v, [spl]) > 0
                x1i = plsc.load_gather(x1_v, [spl])
                y1i = plsc.load_gather(y1_v, [spl])
                x2i = plsc.load_gather(x2_v, [spl])
                y2i = plsc.load_gather(y2_v, [spl])
                ari = plsc.load_gather(ar_v, [spl])
                for c in range(cb, _NCV):
                    sl = pl.ds(c * 16, 16)
                    xx1 = jnp.maximum(x1i, x1_v[sl])
                    yy1 = jnp.maximum(y1i, y1_v[sl])
                    xx2 = jnp.minimum(x2i, x2_v[sl])
                    yy2 = jnp.minimum(y2i, y2_v[sl])
                    w = jnp.maximum(xx2 - xx1, jnp.float32(0.0))
                    h = jnp.maximum(yy2 - yy1, jnp.float32(0.0))
                    inter = w * h
                    union = ari + ar_v[sl] - inter
                    sup = inter > _NMS_THRESH * jnp.maximum(
                        union, jnp.float32(1e-12))
                    if c == cb:
                        sup = jnp.logical_and(sup, (c * 16 + lane) > i)
                    sup = jnp.logical_and(sup, ki)
                    keep_v[sl] = jnp.where(sup, 0, keep_v[sl])
                return 0

            lax.fori_loop(0, 16, nms_body, 0)

        def zero_out(j, _):
            outb_v[pl.ds(j * 16, 16)] = jnp.zeros((16,), jnp.float32)
            return 0

        lax.fori_loop(0, _OUTROW // 16, zero_out, 0)
        cnt = jnp.int32(0)
        for c in range(_NCV):
            sl = pl.ds(c * 16, 16)
            kc = keep_v[sl]
            csum = plsc.cumsum(kc)
            pos = cnt + csum - kc
            m = kc > 0
            base = pos * 5
            plsc.store_scatter(outb_v, [base], scs_v[sl], mask=m)
            plsc.store_scatter(outb_v, [base + 1], x1_v[sl], mask=m)
            plsc.store_scatter(outb_v, [base + 2], y1_v[sl], mask=m)
            plsc.store_scatter(outb_v, [base + 3], x2_v[sl], mask=m)
            plsc.store_scatter(outb_v, [base + 4], y2_v[sl], mask=m)
            cnt = cnt + jnp.sum(kc)
        pltpu.sync_copy(outb_v, out_ref.at[u])

    def round_body(r, _):
        u = wid + r * 32

        @pl.when(u < _NUNITS)
        def _():
            process(u)

        return 0

    lax.fori_loop(0, 3, round_body, 0)


@jax.jit
def kernel(loc_data, conf_data, prior_data):
    conf_t = jnp.transpose(conf_data, (0, 2, 1))[:, 1:, :].reshape(
        _NUNITS, _NUM_PRIORS)
    tbl = jnp.concatenate(
        [loc_data.reshape(_BATCH * _NUM_PRIORS, 4),
         jnp.tile(prior_data, (_BATCH, 1))], axis=1)

    mesh = plsc.VectorSubcoreMesh(core_axis_name="c", subcore_axis_name="s")
    out = pl.kernel(
        _sc_detect,
        out_type=jax.ShapeDtypeStruct((_NUNITS, _OUTROW), jnp.float32),
        mesh=mesh,
        compiler_params=pltpu.CompilerParams(
            needs_layout_passes=False, use_tc_tiling_on_sc=False),
        scratch_types=[
            pltpu.VMEM((_NPAD,), jnp.float32),
            pltpu.VMEM((_NPAD,), jnp.int32),
            pltpu.VMEM((_NPAD,), jnp.int32),
            pltpu.VMEM((_NPAD,), jnp.int32),
            pltpu.VMEM((4096,), jnp.int32),
            pltpu.VMEM((224,), jnp.int32),
            pltpu.VMEM((224,), jnp.int32),
            pltpu.VMEM((_KPAD,), jnp.int32),
            pltpu.VMEM((_KPAD,), jnp.int32),
            pltpu.VMEM((_KPAD,), jnp.int32),
            pltpu.VMEM((2, 104), jnp.int32),
            pltpu.VMEM((_KPAD, 8), jnp.float32),
            pltpu.VMEM((_KPAD,), jnp.float32),
            pltpu.VMEM((_KPAD,), jnp.float32),
            pltpu.VMEM((_KPAD,), jnp.float32),
            pltpu.VMEM((_KPAD,), jnp.float32),
            pltpu.VMEM((_KPAD,), jnp.float32),
            pltpu.VMEM((_KPAD,), jnp.float32),
            pltpu.VMEM((_KPAD,), jnp.int32),
            pltpu.VMEM((_OUTROW,), jnp.float32),
            pltpu.SemaphoreType.DMA,
        ],
    )(conf_t, tbl)

    dets = out[:, : _TOP_K * 5].reshape(_BATCH, _NUM_CLASSES - 1, _TOP_K, 5)
    zeros0 = jnp.zeros((_BATCH, 1, _TOP_K, 5), jnp.float32)
    return jnp.concatenate([zeros0, dets], axis=1)
